# baseline (speedup 1.0000x reference)
import jax
import jax.numpy as jnp
import numpy as np
from jax import lax
from jax.experimental import pallas as pl
from jax.experimental.pallas import tpu as pltpu, tpu_sc as plsc

N = 5000
C = 46
QC = 65
SCORE_THRESH = 0.05
NMS_THRESH = 0.5
DETS = 100
PRE_NMS = 1000
IMG_W = 800.0
IMG_H = 800.0
BBOX_XFORM_CLIP = float(np.log(1000.0 / 16.0))
K = 1024
CHUNK = 128
NTOT = 230400
NTILE = 16
TSZ = NTOT // NTILE
SC_CAP = 256
SENT_SCORE = -2e9
SENT_IDX = 1 << 20
BS_ITERS = 26


def _score_decode_kernel(cl_ref, ql_ref, dx_ref, dy_ref, dw_ref, dh_ref,
                         px1_ref, py1_ref, px2_ref, py2_ref,
                         masked_ref, quant_ref, x1_ref, y1_ref, x2_ref, y2_ref,
                         tlo_ref):
    cl = cl_ref[...]
    m = jnp.max(cl, axis=1, keepdims=True)
    e = jnp.exp(cl - m)
    probs = e / jnp.sum(e, axis=1, keepdims=True)

    ql = ql_ref[...]
    qm = jnp.max(ql, axis=1, keepdims=True)
    qe = jnp.exp(ql - qm)
    qp = qe / jnp.sum(qe, axis=1, keepdims=True)
    qpm = jnp.max(qp, axis=1, keepdims=True)
    qiota = lax.broadcasted_iota(jnp.int32, (N, QC), 1)
    quant_ref[...] = jnp.min(jnp.where(qp >= qpm, qiota, QC), axis=1,
                             keepdims=True)

    px1 = px1_ref[...]; py1 = py1_ref[...]
    px2 = px2_ref[...]; py2 = py2_ref[...]
    widths = px2 - px1
    heights = py2 - py1
    ctr_x = px1 + 0.5 * widths
    ctr_y = py1 + 0.5 * heights
    dx = dx_ref[...] / 10.0
    dy = dy_ref[...] / 10.0
    dw = jnp.minimum(dw_ref[...] / 5.0, BBOX_XFORM_CLIP)
    dh = jnp.minimum(dh_ref[...] / 5.0, BBOX_XFORM_CLIP)
    pcx = dx * widths + ctr_x
    pcy = dy * heights + ctr_y
    pw = jnp.exp(dw) * widths
    ph = jnp.exp(dh) * heights
    x1 = jnp.clip(pcx - 0.5 * pw, 0.0, IMG_W)
    y1 = jnp.clip(pcy - 0.5 * ph, 0.0, IMG_H)
    x2 = jnp.clip(pcx + 0.5 * pw, 0.0, IMG_W)
    y2 = jnp.clip(pcy + 0.5 * ph, 0.0, IMG_H)
    x1_ref[...] = x1
    y1_ref[...] = y1
    x2_ref[...] = x2
    y2_ref[...] = y2

    w = x2 - x1
    h = y2 - y1
    ciota = lax.broadcasted_iota(jnp.int32, (N, C), 1)
    valid = ((probs > SCORE_THRESH) & (w >= 0.01) & (h >= 0.01) & (ciota > 0))
    masked = jnp.where(valid, probs, -1e9)
    masked_ref[...] = masked

    def bs(_, lohi):
        lo, hi = lohi
        mid = 0.5 * (lo + hi)
        cnt = jnp.sum((masked > mid).astype(jnp.float32))
        ge = cnt >= PRE_NMS
        return (jnp.where(ge, mid, lo), jnp.where(ge, hi, mid))

    lo, hi = lax.fori_loop(0, BS_ITERS, bs,
                           (jnp.float32(SCORE_THRESH), jnp.float32(1.0)))
    tlo_ref[...] = jnp.broadcast_to(lo, (1, 1))


def _score_decode(class_logits, quantity_logits, box_regression, proposals):
    regr = box_regression.reshape(N, C, 4)
    dx = regr[..., 0]
    dy = regr[..., 1]
    dw = regr[..., 2]
    dh = regr[..., 3]
    px1 = proposals[:, 0:1]
    py1 = proposals[:, 1:2]
    px2 = proposals[:, 2:3]
    py2 = proposals[:, 3:4]
    f32 = jnp.float32
    out_shapes = (
        jax.ShapeDtypeStruct((N, C), f32),
        jax.ShapeDtypeStruct((N, 1), jnp.int32),
        jax.ShapeDtypeStruct((N, C), f32),
        jax.ShapeDtypeStruct((N, C), f32),
        jax.ShapeDtypeStruct((N, C), f32),
        jax.ShapeDtypeStruct((N, C), f32),
        jax.ShapeDtypeStruct((1, 1), f32),
    )
    return pl.pallas_call(
        _score_decode_kernel,
        out_shape=out_shapes,
    )(class_logits, quantity_logits, dx, dy, dw, dh, px1, py1, px2, py2)


def _compact_body(scores_hbm, tlo_hbm, out_s_hbm, out_i_hbm,
                  svmem, ls, li, tlov):
    w = lax.axis_index("s")
    i16 = lax.broadcasted_iota(jnp.int32, (16,), 0)

    pltpu.sync_copy(scores_hbm.at[pl.ds(w * TSZ, TSZ)], svmem)
    pltpu.sync_copy(tlo_hbm, tlov)
    tlo = tlov[...]

    sentv = jnp.full((16,), SENT_SCORE, jnp.float32)
    senti = jnp.full((16,), SENT_IDX, jnp.int32)
    for b in range((SC_CAP + 16) // 16):
        ls[pl.ds(16 * b, 16)] = sentv
        li[pl.ds(16 * b, 16)] = senti

    def chunk(k, cnt_vec):
        s = svmem[pl.ds(16 * k, 16)]
        mask = s > tlo

        def hit(cnt_vec):
            mi = mask.astype(jnp.int32)
            key = (mi << 8) | i16
            idxv = w * TSZ + 16 * k + i16
            _, ss = plsc.sort_key_val(key, s, descending=True)
            _, si = plsc.sort_key_val(key, idxv, descending=True)
            pos = cnt_vec + i16
            plsc.store_scatter(ls, [pos], ss)
            plsc.store_scatter(li, [pos], si)
            pc = plsc.all_reduce_population_count(mask)
            return jnp.minimum(cnt_vec + pc, SC_CAP)

        return lax.cond(jnp.any(mask), hit, lambda cv: cv, cnt_vec)

    cnt_vec = lax.fori_loop(0, TSZ // 16, chunk, jnp.zeros((16,), jnp.int32))
    pos = cnt_vec + i16
    plsc.store_scatter(ls, [pos], sentv)
    plsc.store_scatter(li, [pos], senti)

    pltpu.sync_copy(ls.at[pl.ds(0, SC_CAP)], out_s_hbm.at[w])
    pltpu.sync_copy(li.at[pl.ds(0, SC_CAP)], out_i_hbm.at[w])


def _compact(scores_flat, tlo16):
    mesh = plsc.VectorSubcoreMesh(core_axis_name="c", subcore_axis_name="s",
                                  num_cores=1)
    f = pl.kernel(
        _compact_body,
        out_type=(jax.ShapeDtypeStruct((NTILE, SC_CAP), jnp.float32),
                  jax.ShapeDtypeStruct((NTILE, SC_CAP), jnp.int32)),
        mesh=mesh,
        scratch_types=[
            pltpu.VMEM((TSZ,), jnp.float32),
            pltpu.VMEM((SC_CAP + 16,), jnp.float32),
            pltpu.VMEM((SC_CAP + 16,), jnp.int32),
            pltpu.VMEM((16,), jnp.float32),
        ],
        compiler_params=pltpu.CompilerParams(needs_layout_passes=False),
    )
    return f(scores_flat, tlo16)


def _sort_pairs_desc(s, idx):
    rows = s.shape[0]
    n = rows * 128
    ri = lax.broadcasted_iota(jnp.int32, (rows, 128), 0)
    ci = lax.broadcasted_iota(jnp.int32, (rows, 128), 1)
    fi = ri * 128 + ci
    k = 2
    while k <= n:
        j = k // 2
        while j >= 1:
            if j < 128:
                ps_a = jnp.roll(s, -j, axis=1)
                ps_b = jnp.roll(s, j, axis=1)
                pi_a = jnp.roll(idx, -j, axis=1)
                pi_b = jnp.roll(idx, j, axis=1)
            else:
                jr = j // 128
                ps_a = jnp.roll(s, -jr, axis=0)
                ps_b = jnp.roll(s, jr, axis=0)
                pi_a = jnp.roll(idx, -jr, axis=0)
                pi_b = jnp.roll(idx, jr, axis=0)
            low = (fi & j) == 0
            ps = jnp.where(low, ps_a, ps_b)
            pi = jnp.where(low, pi_a, pi_b)
            b_own = (s > ps) | ((s == ps) & (idx < pi))
            dirdesc = (fi & k) == 0
            sel = (low == dirdesc) == b_own
            s = jnp.where(sel, s, ps)
            idx = jnp.where(sel, idx, pi)
            j //= 2
        k *= 2
    return s, idx


def _sort_kernel(s_ref, i_ref, os_ref, oi_ref):
    s, idx = _sort_pairs_desc(s_ref[...], i_ref[...])
    os_ref[...] = s
    oi_ref[...] = idx


def _sort_pairs(s2d, i2d):
    return pl.pallas_call(
        _sort_kernel,
        out_shape=(jax.ShapeDtypeStruct(s2d.shape, s2d.dtype),
                   jax.ShapeDtypeStruct(i2d.shape, i2d.dtype)),
    )(s2d, i2d)


def _gather_body(idx_hbm, x1_hbm, y1_hbm, x2_hbm, y2_hbm, q_hbm,
                 ox1, oy1, ox2, oy2, oq, ol, onn,
                 idxv, nv, lv, tmpf, tmpi, sem):
    w = lax.axis_index("s")
    gper = K // NTILE
    base = w * gper
    pltpu.sync_copy(idx_hbm.at[pl.ds(base, gper)], idxv)
    for t in range(gper // 16):
        v = idxv[pl.ds(16 * t, 16)]
        v = jnp.minimum(v, N * C - 1)
        n = v // C
        idxv[pl.ds(16 * t, 16)] = v
        nv[pl.ds(16 * t, 16)] = n
        lv[pl.ds(16 * t, 16)] = v - n * C
    for table, out in ((x1_hbm, ox1), (y1_hbm, oy1),
                       (x2_hbm, ox2), (y2_hbm, oy2)):
        pltpu.async_copy(table.at[idxv], tmpf, sem).wait()
        pltpu.sync_copy(tmpf, out.at[pl.ds(base, gper)])
    pltpu.async_copy(q_hbm.at[nv], tmpi, sem).wait()
    pltpu.sync_copy(tmpi, oq.at[pl.ds(base, gper)])
    pltpu.sync_copy(lv, ol.at[pl.ds(base, gper)])
    pltpu.sync_copy(nv, onn.at[pl.ds(base, gper)])


def _gather_cands(idx1024, x1f, y1f, x2f, y2f, quant_flat):
    mesh = plsc.VectorSubcoreMesh(core_axis_name="c", subcore_axis_name="s",
                                  num_cores=1)
    f32 = jnp.float32
    i32 = jnp.int32
    gper = K // NTILE
    f = pl.kernel(
        _gather_body,
        out_type=(jax.ShapeDtypeStruct((K,), f32),
                  jax.ShapeDtypeStruct((K,), f32),
                  jax.ShapeDtypeStruct((K,), f32),
                  jax.ShapeDtypeStruct((K,), f32),
                  jax.ShapeDtypeStruct((K,), i32),
                  jax.ShapeDtypeStruct((K,), i32),
                  jax.ShapeDtypeStruct((K,), i32)),
        mesh=mesh,
        scratch_types=[
            pltpu.VMEM((gper,), i32),
            pltpu.VMEM((gper,), i32),
            pltpu.VMEM((gper,), i32),
            pltpu.VMEM((gper,), f32),
            pltpu.VMEM((gper,), i32),
            pltpu.SemaphoreType.DMA,
        ],
        compiler_params=pltpu.CompilerParams(needs_layout_passes=False),
    )
    return f(idx1024, x1f, y1f, x2f, y2f, quant_flat)


def _nms_kernel(xi1_ref, yi1_ref, xi2_ref, yi2_ref, li_ref,
                xj1_ref, yj1_ref, xj2_ref, yj2_ref, lj_ref, sc8_ref,
                fs_ref, fp_ref, suploc_ref):
    f32 = jnp.float32
    offi = li_ref[...].astype(f32) * (IMG_W + 1.0)
    offj = lj_ref[...].astype(f32) * (IMG_W + 1.0)
    ax1 = xi1_ref[...] + offi
    ay1 = yi1_ref[...] + offi
    ax2 = xi2_ref[...] + offi
    ay2 = yi2_ref[...] + offi
    bx1 = xj1_ref[...] + offj
    by1 = yj1_ref[...] + offj
    bx2 = xj2_ref[...] + offj
    by2 = yj2_ref[...] + offj
    area_i = (ax2 - ax1) * (ay2 - ay1)
    area_j = (bx2 - bx1) * (by2 - by1)

    supacc = jnp.zeros((1, K), dtype=f32)
    i128 = lax.broadcasted_iota(jnp.int32, (1, CHUNK), 1)
    kcs = []

    for c in range(K // CHUNK):
        lo = c * CHUNK
        cx1 = lax.slice(ax1, (lo, 0), (lo + CHUNK, 1))
        cy1 = lax.slice(ay1, (lo, 0), (lo + CHUNK, 1))
        cx2 = lax.slice(ax2, (lo, 0), (lo + CHUNK, 1))
        cy2 = lax.slice(ay2, (lo, 0), (lo + CHUNK, 1))
        carea = lax.slice(area_i, (lo, 0), (lo + CHUNK, 1))
        ltx = jnp.maximum(cx1, bx1)
        lty = jnp.maximum(cy1, by1)
        rbx = jnp.minimum(cx2, bx2)
        rby = jnp.minimum(cy2, by2)
        wx = jnp.clip(rbx - ltx, 0.0, None)
        wy = jnp.clip(rby - lty, 0.0, None)
        inter = wx * wy
        iou = inter / (carea + area_j - inter + 1e-9)
        supf = (iou > NMS_THRESH).astype(f32)
        suploc_ref[...] = lax.slice(supf, (0, lo), (CHUNK, lo + CHUNK))

        kc = (lax.slice(supacc, (0, lo), (1, lo + CHUNK)) <= 0.5).astype(f32)

        def body(i, kc):
            row = suploc_ref[pl.ds(i, 1), :]
            ki = jnp.max(jnp.where(i128 == i, kc, 0.0))
            return kc * (1.0 - row * (i128 > i).astype(f32) * ki)

        kc = lax.fori_loop(0, CHUNK, body, kc)
        kcs.append(kc)
        supv = lax.dot_general(kc, supf, (((1,), (0,)), ((), ())),
                               preferred_element_type=f32)
        supacc = supacc + supv

    rows = [jnp.where(kcs[c] > 0.5, sc8_ref[c:c + 1, :], -1e9)
            for c in range(K // CHUNK)]
    fm = jnp.concatenate(rows, axis=0)
    ri = lax.broadcasted_iota(jnp.int32, (K // CHUNK, CHUNK), 0)
    ci = lax.broadcasted_iota(jnp.int32, (K // CHUNK, CHUNK), 1)
    fs, fp = _sort_pairs_desc(fm, ri * CHUNK + ci)
    fs_ref[...] = fs
    fp_ref[...] = fp


def _nms(cx1, cy1, cx2, cy2, labels, scores):
    f32 = jnp.float32
    xi1 = cx1.reshape(K, 1)
    yi1 = cy1.reshape(K, 1)
    xi2 = cx2.reshape(K, 1)
    yi2 = cy2.reshape(K, 1)
    li = labels.reshape(K, 1)
    sc8 = scores.reshape(K // CHUNK, CHUNK)
    fs, fp = pl.pallas_call(
        _nms_kernel,
        out_shape=(jax.ShapeDtypeStruct((K // CHUNK, CHUNK), f32),
                   jax.ShapeDtypeStruct((K // CHUNK, CHUNK), jnp.int32)),
        scratch_shapes=[pltpu.VMEM((CHUNK, CHUNK), f32)],
    )(xi1, yi1, xi2, yi2, li, cx1.reshape(1, K), cy1.reshape(1, K),
      cx2.reshape(1, K), cy2.reshape(1, K), li.reshape(1, K), sc8)
    return fs, fp


def kernel(class_logits, quantity_logits, box_features, box_regression,
           proposals):
    masked, quant, x1, y1, x2, y2, tlo = _score_decode(
        class_logits, quantity_logits, box_regression, proposals)

    flat = jnp.pad(masked.reshape(-1), (0, NTOT - N * C),
                   constant_values=-1e9)
    tlo16 = jnp.broadcast_to(tlo.reshape(1), (16,))
    cs, ci = _compact(flat, tlo16)

    ss, si = _sort_pairs(cs.reshape(32, 128), ci.reshape(32, 128))
    top_scores = ss.reshape(-1)[:PRE_NMS]
    top_idx = si.reshape(-1)[:PRE_NMS]

    idx1024 = jnp.pad(top_idx, (0, K - PRE_NMS))
    sc1024 = jnp.pad(top_scores, (0, K - PRE_NMS), constant_values=-1e9)
    cx1, cy1, cx2, cy2, quants, labels, n_idx = _gather_cands(
        idx1024, x1.reshape(-1), y1.reshape(-1), x2.reshape(-1),
        y2.reshape(-1), quant.reshape(-1))

    fs, fp = _nms(cx1, cy1, cx2, cy2, labels, sc1024)
    out_scores = fs.reshape(-1)[:DETS]
    sel = fp.reshape(-1)[:DETS]

    out_boxes = jnp.stack([jnp.take(cx1, sel), jnp.take(cy1, sel),
                           jnp.take(cx2, sel), jnp.take(cy2, sel)], axis=-1)
    out_labels = jnp.take(labels, sel)
    out_quants = jnp.take(quants, sel)
    bidx = jnp.take(n_idx, sel)
    out_feats = jnp.take(box_features, bidx, axis=0)
    return out_boxes, out_scores, out_labels, out_quants, out_feats


# speedup vs baseline: 1.0640x; 1.0640x over previous
import jax
import jax.numpy as jnp
import numpy as np
from jax import lax
from jax.experimental import pallas as pl
from jax.experimental.pallas import tpu as pltpu, tpu_sc as plsc

N = 5000
C = 46
QC = 65
SCORE_THRESH = 0.05
NMS_THRESH = 0.5
DETS = 100
PRE_NMS = 1000
IMG_W = 800.0
IMG_H = 800.0
BBOX_XFORM_CLIP = float(np.log(1000.0 / 16.0))
K = 1024
CHUNK = 128
NTOT = 230400
NTILE = 32
TSZ = NTOT // NTILE
SC_CAP = 128
SENT_SCORE = -2e9
SENT_IDX = 1 << 20
BS_ITERS = 26


def _score_decode_kernel(cl_ref, ql_ref, dx_ref, dy_ref, dw_ref, dh_ref,
                         px1_ref, py1_ref, px2_ref, py2_ref,
                         masked_ref, quant_ref, x1_ref, y1_ref, x2_ref, y2_ref,
                         tlo_ref):
    cl = cl_ref[...]
    m = jnp.max(cl, axis=1, keepdims=True)
    e = jnp.exp(cl - m)
    probs = e / jnp.sum(e, axis=1, keepdims=True)

    ql = ql_ref[...]
    qm = jnp.max(ql, axis=1, keepdims=True)
    qe = jnp.exp(ql - qm)
    qp = qe / jnp.sum(qe, axis=1, keepdims=True)
    qpm = jnp.max(qp, axis=1, keepdims=True)
    qiota = lax.broadcasted_iota(jnp.int32, (N, QC), 1)
    quant_ref[...] = jnp.min(jnp.where(qp >= qpm, qiota, QC), axis=1,
                             keepdims=True)

    px1 = px1_ref[...]; py1 = py1_ref[...]
    px2 = px2_ref[...]; py2 = py2_ref[...]
    widths = px2 - px1
    heights = py2 - py1
    ctr_x = px1 + 0.5 * widths
    ctr_y = py1 + 0.5 * heights
    dx = dx_ref[...] / 10.0
    dy = dy_ref[...] / 10.0
    dw = jnp.minimum(dw_ref[...] / 5.0, BBOX_XFORM_CLIP)
    dh = jnp.minimum(dh_ref[...] / 5.0, BBOX_XFORM_CLIP)
    pcx = dx * widths + ctr_x
    pcy = dy * heights + ctr_y
    pw = jnp.exp(dw) * widths
    ph = jnp.exp(dh) * heights
    x1 = jnp.clip(pcx - 0.5 * pw, 0.0, IMG_W)
    y1 = jnp.clip(pcy - 0.5 * ph, 0.0, IMG_H)
    x2 = jnp.clip(pcx + 0.5 * pw, 0.0, IMG_W)
    y2 = jnp.clip(pcy + 0.5 * ph, 0.0, IMG_H)
    x1_ref[...] = x1
    y1_ref[...] = y1
    x2_ref[...] = x2
    y2_ref[...] = y2

    w = x2 - x1
    h = y2 - y1
    ciota = lax.broadcasted_iota(jnp.int32, (N, C), 1)
    valid = ((probs > SCORE_THRESH) & (w >= 0.01) & (h >= 0.01) & (ciota > 0))
    masked = jnp.where(valid, probs, -1e9)
    masked_ref[...] = masked

    def bs(_, lohi):
        lo, hi = lohi
        mid = 0.5 * (lo + hi)
        cnt = jnp.sum((masked > mid).astype(jnp.float32))
        ge = cnt >= PRE_NMS
        return (jnp.where(ge, mid, lo), jnp.where(ge, hi, mid))

    lo, hi = lax.fori_loop(0, BS_ITERS, bs,
                           (jnp.float32(SCORE_THRESH), jnp.float32(1.0)))
    tlo_ref[...] = jnp.broadcast_to(lo, (1, 1))


def _score_decode(class_logits, quantity_logits, box_regression, proposals):
    regr = box_regression.reshape(N, C, 4)
    dx = regr[..., 0]
    dy = regr[..., 1]
    dw = regr[..., 2]
    dh = regr[..., 3]
    px1 = proposals[:, 0:1]
    py1 = proposals[:, 1:2]
    px2 = proposals[:, 2:3]
    py2 = proposals[:, 3:4]
    f32 = jnp.float32
    out_shapes = (
        jax.ShapeDtypeStruct((N, C), f32),
        jax.ShapeDtypeStruct((N, 1), jnp.int32),
        jax.ShapeDtypeStruct((N, C), f32),
        jax.ShapeDtypeStruct((N, C), f32),
        jax.ShapeDtypeStruct((N, C), f32),
        jax.ShapeDtypeStruct((N, C), f32),
        jax.ShapeDtypeStruct((1, 1), f32),
    )
    return pl.pallas_call(
        _score_decode_kernel,
        out_shape=out_shapes,
    )(class_logits, quantity_logits, dx, dy, dw, dh, px1, py1, px2, py2)


def _compact_body(scores_hbm, tlo_hbm, out_s_hbm, out_i_hbm,
                  svmem, ls, li, tlov):
    w = lax.axis_index("s") * 2 + lax.axis_index("c")
    i16 = lax.broadcasted_iota(jnp.int32, (16,), 0)

    pltpu.sync_copy(scores_hbm.at[pl.ds(w * TSZ, TSZ)], svmem)
    pltpu.sync_copy(tlo_hbm, tlov)
    tlo = tlov[...]

    sentv = jnp.full((16,), SENT_SCORE, jnp.float32)
    senti = jnp.full((16,), SENT_IDX, jnp.int32)
    for b in range((SC_CAP + 16) // 16):
        ls[pl.ds(16 * b, 16)] = sentv
        li[pl.ds(16 * b, 16)] = senti

    def chunk(k, cnt_vec):
        s = svmem[pl.ds(16 * k, 16)]
        mask = s > tlo
        mi = mask.astype(jnp.int32)
        key = (mi << 8) | i16
        idxv = w * TSZ + 16 * k + i16
        _, ss = plsc.sort_key_val(key, s, descending=True)
        _, si = plsc.sort_key_val(key, idxv, descending=True)
        pos = cnt_vec + i16
        plsc.store_scatter(ls, [pos], ss)
        plsc.store_scatter(li, [pos], si)
        pc = plsc.all_reduce_population_count(mask)
        return jnp.minimum(cnt_vec + pc, SC_CAP)

    cnt_vec = lax.fori_loop(0, TSZ // 16, chunk, jnp.zeros((16,), jnp.int32))
    pos = cnt_vec + i16
    plsc.store_scatter(ls, [pos], sentv)
    plsc.store_scatter(li, [pos], senti)

    pltpu.sync_copy(ls.at[pl.ds(0, SC_CAP)], out_s_hbm.at[w])
    pltpu.sync_copy(li.at[pl.ds(0, SC_CAP)], out_i_hbm.at[w])


def _compact(scores_flat, tlo16):
    mesh = plsc.VectorSubcoreMesh(core_axis_name="c", subcore_axis_name="s",
                                  num_cores=2)
    f = pl.kernel(
        _compact_body,
        out_type=(jax.ShapeDtypeStruct((NTILE, SC_CAP), jnp.float32),
                  jax.ShapeDtypeStruct((NTILE, SC_CAP), jnp.int32)),
        mesh=mesh,
        scratch_types=[
            pltpu.VMEM((TSZ,), jnp.float32),
            pltpu.VMEM((SC_CAP + 16,), jnp.float32),
            pltpu.VMEM((SC_CAP + 16,), jnp.int32),
            pltpu.VMEM((16,), jnp.float32),
        ],
        compiler_params=pltpu.CompilerParams(needs_layout_passes=False),
    )
    return f(scores_flat, tlo16)


def _sort_pairs_desc(s, idx):
    rows = s.shape[0]
    n = rows * 128
    ri = lax.broadcasted_iota(jnp.int32, (rows, 128), 0)
    ci = lax.broadcasted_iota(jnp.int32, (rows, 128), 1)
    fi = ri * 128 + ci
    k = 2
    while k <= n:
        j = k // 2
        while j >= 1:
            if j < 128:
                ps_a = jnp.roll(s, -j, axis=1)
                ps_b = jnp.roll(s, j, axis=1)
                pi_a = jnp.roll(idx, -j, axis=1)
                pi_b = jnp.roll(idx, j, axis=1)
            else:
                jr = j // 128
                ps_a = jnp.roll(s, -jr, axis=0)
                ps_b = jnp.roll(s, jr, axis=0)
                pi_a = jnp.roll(idx, -jr, axis=0)
                pi_b = jnp.roll(idx, jr, axis=0)
            low = (fi & j) == 0
            ps = jnp.where(low, ps_a, ps_b)
            pi = jnp.where(low, pi_a, pi_b)
            b_own = (s > ps) | ((s == ps) & (idx < pi))
            dirdesc = (fi & k) == 0
            sel = (low == dirdesc) == b_own
            s = jnp.where(sel, s, ps)
            idx = jnp.where(sel, idx, pi)
            j //= 2
        k *= 2
    return s, idx


def _sort_kernel(s_ref, i_ref, os_ref, oi_ref):
    s, idx = _sort_pairs_desc(s_ref[...], i_ref[...])
    os_ref[...] = s
    oi_ref[...] = idx


def _sort_pairs(s2d, i2d):
    return pl.pallas_call(
        _sort_kernel,
        out_shape=(jax.ShapeDtypeStruct(s2d.shape, s2d.dtype),
                   jax.ShapeDtypeStruct(i2d.shape, i2d.dtype)),
    )(s2d, i2d)


def _gather_body(idx_hbm, x1_hbm, y1_hbm, x2_hbm, y2_hbm, q_hbm,
                 ox1, oy1, ox2, oy2, oq, ol, onn,
                 idxv, nv, lv, tmpf, tmpi, sem):
    w = lax.axis_index("s") * 2 + lax.axis_index("c")
    gper = K // NTILE
    base = w * gper
    pltpu.sync_copy(idx_hbm.at[pl.ds(base, gper)], idxv)
    for t in range(gper // 16):
        v = idxv[pl.ds(16 * t, 16)]
        v = jnp.minimum(v, N * C - 1)
        n = v // C
        idxv[pl.ds(16 * t, 16)] = v
        nv[pl.ds(16 * t, 16)] = n
        lv[pl.ds(16 * t, 16)] = v - n * C
    for table, out in ((x1_hbm, ox1), (y1_hbm, oy1),
                       (x2_hbm, ox2), (y2_hbm, oy2)):
        pltpu.async_copy(table.at[idxv], tmpf, sem).wait()
        pltpu.sync_copy(tmpf, out.at[pl.ds(base, gper)])
    pltpu.async_copy(q_hbm.at[nv], tmpi, sem).wait()
    pltpu.sync_copy(tmpi, oq.at[pl.ds(base, gper)])
    pltpu.sync_copy(lv, ol.at[pl.ds(base, gper)])
    pltpu.sync_copy(nv, onn.at[pl.ds(base, gper)])


def _gather_cands(idx1024, x1f, y1f, x2f, y2f, quant_flat):
    mesh = plsc.VectorSubcoreMesh(core_axis_name="c", subcore_axis_name="s",
                                  num_cores=2)
    f32 = jnp.float32
    i32 = jnp.int32
    gper = K // NTILE
    f = pl.kernel(
        _gather_body,
        out_type=(jax.ShapeDtypeStruct((K,), f32),
                  jax.ShapeDtypeStruct((K,), f32),
                  jax.ShapeDtypeStruct((K,), f32),
                  jax.ShapeDtypeStruct((K,), f32),
                  jax.ShapeDtypeStruct((K,), i32),
                  jax.ShapeDtypeStruct((K,), i32),
                  jax.ShapeDtypeStruct((K,), i32)),
        mesh=mesh,
        scratch_types=[
            pltpu.VMEM((gper,), i32),
            pltpu.VMEM((gper,), i32),
            pltpu.VMEM((gper,), i32),
            pltpu.VMEM((gper,), f32),
            pltpu.VMEM((gper,), i32),
            pltpu.SemaphoreType.DMA,
        ],
        compiler_params=pltpu.CompilerParams(needs_layout_passes=False),
    )
    return f(idx1024, x1f, y1f, x2f, y2f, quant_flat)


def _nms_kernel(xi1_ref, yi1_ref, xi2_ref, yi2_ref, li_ref,
                xj1_ref, yj1_ref, xj2_ref, yj2_ref, lj_ref, sc8_ref,
                fs_ref, fp_ref, suploc_ref):
    f32 = jnp.float32
    offi = li_ref[...].astype(f32) * (IMG_W + 1.0)
    offj = lj_ref[...].astype(f32) * (IMG_W + 1.0)
    ax1 = xi1_ref[...] + offi
    ay1 = yi1_ref[...] + offi
    ax2 = xi2_ref[...] + offi
    ay2 = yi2_ref[...] + offi
    bx1 = xj1_ref[...] + offj
    by1 = yj1_ref[...] + offj
    bx2 = xj2_ref[...] + offj
    by2 = yj2_ref[...] + offj
    area_i = (ax2 - ax1) * (ay2 - ay1)
    area_j = (bx2 - bx1) * (by2 - by1)

    supacc = jnp.zeros((1, K), dtype=f32)
    i128 = lax.broadcasted_iota(jnp.int32, (1, CHUNK), 1)
    kcs = []

    for c in range(K // CHUNK):
        lo = c * CHUNK
        cx1 = lax.slice(ax1, (lo, 0), (lo + CHUNK, 1))
        cy1 = lax.slice(ay1, (lo, 0), (lo + CHUNK, 1))
        cx2 = lax.slice(ax2, (lo, 0), (lo + CHUNK, 1))
        cy2 = lax.slice(ay2, (lo, 0), (lo + CHUNK, 1))
        carea = lax.slice(area_i, (lo, 0), (lo + CHUNK, 1))
        ltx = jnp.maximum(cx1, bx1)
        lty = jnp.maximum(cy1, by1)
        rbx = jnp.minimum(cx2, bx2)
        rby = jnp.minimum(cy2, by2)
        wx = jnp.clip(rbx - ltx, 0.0, None)
        wy = jnp.clip(rby - lty, 0.0, None)
        inter = wx * wy
        iou = inter / (carea + area_j - inter + 1e-9)
        supf = (iou > NMS_THRESH).astype(f32)
        suploc_ref[...] = lax.slice(supf, (0, lo), (CHUNK, lo + CHUNK))

        kc = (lax.slice(supacc, (0, lo), (1, lo + CHUNK)) <= 0.5).astype(f32)

        def body(i, kc):
            row = suploc_ref[pl.ds(i, 1), :]
            ki = jnp.max(jnp.where(i128 == i, kc, 0.0))
            return kc * (1.0 - row * (i128 > i).astype(f32) * ki)

        kc = lax.fori_loop(0, CHUNK, body, kc, unroll=8)
        kcs.append(kc)
        supv = lax.dot_general(kc, supf, (((1,), (0,)), ((), ())),
                               preferred_element_type=f32)
        supacc = supacc + supv

    rows = [jnp.where(kcs[c] > 0.5, sc8_ref[c:c + 1, :], -1e9)
            for c in range(K // CHUNK)]
    fm = jnp.concatenate(rows, axis=0)
    ri = lax.broadcasted_iota(jnp.int32, (K // CHUNK, CHUNK), 0)
    ci = lax.broadcasted_iota(jnp.int32, (K // CHUNK, CHUNK), 1)
    fs, fp = _sort_pairs_desc(fm, ri * CHUNK + ci)
    fs_ref[...] = fs
    fp_ref[...] = fp


def _nms(cx1, cy1, cx2, cy2, labels, scores):
    f32 = jnp.float32
    xi1 = cx1.reshape(K, 1)
    yi1 = cy1.reshape(K, 1)
    xi2 = cx2.reshape(K, 1)
    yi2 = cy2.reshape(K, 1)
    li = labels.reshape(K, 1)
    sc8 = scores.reshape(K // CHUNK, CHUNK)
    fs, fp = pl.pallas_call(
        _nms_kernel,
        out_shape=(jax.ShapeDtypeStruct((K // CHUNK, CHUNK), f32),
                   jax.ShapeDtypeStruct((K // CHUNK, CHUNK), jnp.int32)),
        scratch_shapes=[pltpu.VMEM((CHUNK, CHUNK), f32)],
    )(xi1, yi1, xi2, yi2, li, cx1.reshape(1, K), cy1.reshape(1, K),
      cx2.reshape(1, K), cy2.reshape(1, K), li.reshape(1, K), sc8)
    return fs, fp


def kernel(class_logits, quantity_logits, box_features, box_regression,
           proposals):
    masked, quant, x1, y1, x2, y2, tlo = _score_decode(
        class_logits, quantity_logits, box_regression, proposals)

    flat = jnp.pad(masked.reshape(-1), (0, NTOT - N * C),
                   constant_values=-1e9)
    tlo16 = jnp.broadcast_to(tlo.reshape(1), (16,))
    cs, ci = _compact(flat, tlo16)

    ss, si = _sort_pairs(cs.reshape(32, 128), ci.reshape(32, 128))
    top_scores = ss.reshape(-1)[:PRE_NMS]
    top_idx = si.reshape(-1)[:PRE_NMS]

    idx1024 = jnp.pad(top_idx, (0, K - PRE_NMS))
    sc1024 = jnp.pad(top_scores, (0, K - PRE_NMS), constant_values=-1e9)
    cx1, cy1, cx2, cy2, quants, labels, n_idx = _gather_cands(
        idx1024, x1.reshape(-1), y1.reshape(-1), x2.reshape(-1),
        y2.reshape(-1), quant.reshape(-1))

    fs, fp = _nms(cx1, cy1, cx2, cy2, labels, sc1024)
    out_scores = fs.reshape(-1)[:DETS]
    sel = fp.reshape(-1)[:DETS]

    out_boxes = jnp.stack([jnp.take(cx1, sel), jnp.take(cy1, sel),
                           jnp.take(cx2, sel), jnp.take(cy2, sel)], axis=-1)
    out_labels = jnp.take(labels, sel)
    out_quants = jnp.take(quants, sel)
    bidx = jnp.take(n_idx, sel)
    out_feats = jnp.take(box_features, bidx, axis=0)
    return out_boxes, out_scores, out_labels, out_quants, out_feats
